# baseline (speedup 1.0000x reference)
import functools
import math

import jax
import jax.numpy as jnp
from jax import lax
from jax.experimental import pallas as pl
from jax.experimental.pallas import tpu as pltpu
from jax.experimental.pallas import tpu_sc as plsc

N_NODES = 10000
DIM = 128
N_EDGES = 320000
N_LAYERS = 3
N_CLASSES = 10

BLK = 512
N_SUB = 16
CHUNK = 128
CHUNKS_PER_SUB = 160
E_PAD = N_SUB * CHUNKS_PER_SUB * CHUNK
HALF = DIM // 2

N_REAL = [N_NODES >> i for i in range(N_LAYERS)]
N_PAD = [-(-n // BLK) * BLK for n in N_REAL]

INV_SQRT2 = 1.0 / math.sqrt(2.0)


def _make_sc_agg(n_pad: int, shift: int):
  mesh = plsc.VectorSubcoreMesh(core_axis_name="c", subcore_axis_name="s")
  rows_per_sub = n_pad // N_SUB

  @functools.partial(
      pl.kernel,
      mesh=mesh,
      compiler_params=pltpu.CompilerParams(use_tc_tiling_on_sc=False),
      out_type=jax.ShapeDtypeStruct((2, n_pad, HALF), jnp.float32),
      scratch_types=[
          pltpu.VMEM((CHUNKS_PER_SUB, CHUNK), jnp.int32),
          pltpu.VMEM((CHUNKS_PER_SUB, CHUNK), jnp.int32),
          pltpu.VMEM((CHUNK, HALF), jnp.float32),
          pltpu.VMEM((CHUNK, HALF), jnp.float32),
          pltpu.VMEM((32, HALF), jnp.float32),
          pltpu.VMEM_SHARED((n_pad, HALF), jnp.float32),
          pltpu.SemaphoreType.DMA,
          pltpu.SemaphoreType.DMA,
      ],
  )
  def sc_agg(h2_hbm, src_hbm, dst_hbm, out_hbm, src_v, dst_v, rows0, rows1,
             zbuf, agg_sh, gs0, gs1):
    cid = lax.axis_index("c")
    sid = lax.axis_index("s")

    @pl.loop(0, 32)
    def _(r):
      @pl.loop(0, HALF, step=16)
      def _(t):
        zbuf[r, pl.ds(t, 16)] = jnp.zeros((16,), jnp.float32)

    base = sid * rows_per_sub

    @pl.loop(0, rows_per_sub, step=32)
    def _(r):
      pltpu.sync_copy(zbuf, agg_sh.at[pl.ds(base + r, 32)])

    pltpu.sync_copy(src_hbm.at[sid], src_v)
    pltpu.sync_copy(dst_hbm.at[sid], dst_v)

    @pl.loop(0, CHUNKS_PER_SUB)
    def _(j):
      for t in range(CHUNK // 16):
        sl = pl.ds(t * 16, 16)
        s = src_v[j, sl]
        if shift:
          s = lax.shift_right_logical(s, shift)
          dst_v[j, sl] = lax.shift_right_logical(dst_v[j, sl], shift)
        src_v[j, sl] = lax.shift_left(s, 1) | cid

    plsc.subcore_barrier()

    pltpu.async_copy(h2_hbm.at[src_v.at[0]], rows0, gs0)

    @pl.loop(0, CHUNKS_PER_SUB, step=2)
    def _(j):
      pltpu.async_copy(h2_hbm.at[src_v.at[j + 1]], rows1, gs1)
      pltpu.make_async_copy(h2_hbm.at[pl.ds(0, CHUNK)], rows0, gs0).wait()
      pltpu.sync_copy(rows0, agg_sh.at[dst_v.at[j]], add=True)

      @pl.when(j + 2 < CHUNKS_PER_SUB)
      def _():
        pltpu.async_copy(h2_hbm.at[src_v.at[j + 2]], rows0, gs0)

      pltpu.make_async_copy(h2_hbm.at[pl.ds(0, CHUNK)], rows1, gs1).wait()
      pltpu.sync_copy(rows1, agg_sh.at[dst_v.at[j + 1]], add=True)

    plsc.subcore_barrier()

    @pl.loop(0, rows_per_sub, step=32)
    def _(r):
      pltpu.sync_copy(agg_sh.at[pl.ds(base + r, 32)], zbuf)
      pltpu.sync_copy(zbuf, out_hbm.at[cid, pl.ds(base + r, 32)])

  return sc_agg


CHUNKS_FULL = E_PAD // (32 * CHUNK)


def _make_sc_agg_full(n_pad: int, shift: int):
  mesh = plsc.VectorSubcoreMesh(core_axis_name="c", subcore_axis_name="s")
  rows_per_sub = n_pad // N_SUB

  @functools.partial(
      pl.kernel,
      mesh=mesh,
      out_type=jax.ShapeDtypeStruct((2, n_pad, DIM), jnp.float32),
      scratch_types=[
          pltpu.VMEM((CHUNKS_FULL, CHUNK), jnp.int32),
          pltpu.VMEM((CHUNKS_FULL, CHUNK), jnp.int32),
          pltpu.VMEM((CHUNK, DIM), jnp.float32),
          pltpu.VMEM((CHUNK, DIM), jnp.float32),
          pltpu.VMEM((32, DIM), jnp.float32),
          pltpu.VMEM_SHARED((n_pad, DIM), jnp.float32),
          pltpu.SemaphoreType.DMA,
          pltpu.SemaphoreType.DMA,
      ],
  )
  def sc_agg(h_hbm, src_hbm, dst_hbm, out_hbm, src_v, dst_v, rows0, rows1,
             zbuf, agg_sh, gs0, gs1):
    cid = lax.axis_index("c")
    sid = lax.axis_index("s")
    wid = sid * 2 + cid

    @pl.loop(0, 32)
    def _(r):
      @pl.loop(0, DIM, step=16)
      def _(t):
        zbuf[r, pl.ds(t, 16)] = jnp.zeros((16,), jnp.float32)

    base = sid * rows_per_sub

    @pl.loop(0, rows_per_sub, step=32)
    def _(r):
      pltpu.sync_copy(zbuf, agg_sh.at[pl.ds(base + r, 32)])

    pltpu.sync_copy(src_hbm.at[wid], src_v)
    pltpu.sync_copy(dst_hbm.at[wid], dst_v)
    if shift:
      @pl.loop(0, CHUNKS_FULL)
      def _(j):
        for t in range(CHUNK // 16):
          sl = pl.ds(t * 16, 16)
          src_v[j, sl] = lax.shift_right_logical(src_v[j, sl], shift)
          dst_v[j, sl] = lax.shift_right_logical(dst_v[j, sl], shift)

    plsc.subcore_barrier()

    pltpu.async_copy(h_hbm.at[src_v.at[0]], rows0, gs0)

    @pl.loop(0, CHUNKS_FULL, step=2)
    def _(j):
      pltpu.async_copy(h_hbm.at[src_v.at[j + 1]], rows1, gs1)
      pltpu.make_async_copy(h_hbm.at[pl.ds(0, CHUNK)], rows0, gs0).wait()
      pltpu.sync_copy(rows0, agg_sh.at[dst_v.at[j]], add=True)

      @pl.when(j + 2 < CHUNKS_FULL)
      def _():
        pltpu.async_copy(h_hbm.at[src_v.at[j + 2]], rows0, gs0)

      pltpu.make_async_copy(h_hbm.at[pl.ds(0, CHUNK)], rows1, gs1).wait()
      pltpu.sync_copy(rows1, agg_sh.at[dst_v.at[j + 1]], add=True)

    plsc.subcore_barrier()

    @pl.loop(0, rows_per_sub, step=32)
    def _(r):
      pltpu.sync_copy(agg_sh.at[pl.ds(base + r, 32)], zbuf)
      pltpu.sync_copy(zbuf, out_hbm.at[cid, pl.ds(base + r, 32)])

  return sc_agg


def _start_body(n_real, x_ref, w_ref, v_ref, h_ref):
  i = pl.program_id(0)
  z = jnp.dot(x_ref[...], w_ref[...], preferred_element_type=jnp.float32)
  h = jax.nn.relu(v_ref[1:2, :] * (z + v_ref[0:1, :]) + v_ref[2:3, :])
  row = i * BLK + lax.broadcasted_iota(jnp.int32, (BLK, 1), 0)
  h_ref[...] = jnp.where(row < n_real, h, 0.0)


def _make_start(n_pad, n_real):
  grid = n_pad // BLK
  return pl.pallas_call(
      functools.partial(_start_body, n_real),
      grid=(grid,),
      in_specs=[
          pl.BlockSpec((BLK, DIM), lambda i: (i, 0)),
          pl.BlockSpec((DIM, DIM), lambda i: (0, 0)),
          pl.BlockSpec((3, DIM), lambda i: (0, 0)),
      ],
      out_specs=pl.BlockSpec((BLK, DIM), lambda i: (i, 0)),
      out_shape=jax.ShapeDtypeStruct((n_pad, DIM), jnp.float32),
  )


def _mlp_body(n_real, col_split, h_ref, a0_ref, a1_ref, w1_ref, w2_ref,
              v_ref, p_ref, pool_ref, cs_ref):
  i = pl.program_id(0)
  if col_split:
    agg = jnp.concatenate([a0_ref[0], a1_ref[0]], axis=-1)
  else:
    agg = a0_ref[0] + a1_ref[0]
  g = h_ref[...] + agg
  z = jnp.dot(g, w1_ref[...], preferred_element_type=jnp.float32)
  z = jax.nn.relu(v_ref[1:2, :] * (z + v_ref[0:1, :]) + v_ref[2:3, :])
  u = jnp.dot(z, w2_ref[...], preferred_element_type=jnp.float32)
  u = jax.nn.relu(v_ref[4:5, :] * (u + v_ref[3:4, :]) + v_ref[5:6, :])
  row = i * BLK + lax.broadcasted_iota(jnp.int32, (BLK, 1), 0)
  u = jnp.where(row < n_real, u, 0.0)
  pool_ref[...] = jnp.dot(p_ref[...], u, preferred_element_type=jnp.float32)
  cs = jnp.sum(u, axis=0, keepdims=True)

  @pl.when(i == 0)
  def _():
    cs_ref[...] = cs

  @pl.when(i > 0)
  def _():
    cs_ref[...] += cs


def _make_mlp(n_pad, n_real, col_split):
  grid = n_pad // BLK
  aw = HALF if col_split else DIM
  return pl.pallas_call(
      functools.partial(_mlp_body, n_real, col_split),
      grid=(grid,),
      in_specs=[
          pl.BlockSpec((BLK, DIM), lambda i: (i, 0)),
          pl.BlockSpec((1, BLK, aw), lambda i: (0, i, 0)),
          pl.BlockSpec((1, BLK, aw), lambda i: (1, i, 0)),
          pl.BlockSpec((DIM, DIM), lambda i: (0, 0)),
          pl.BlockSpec((DIM, DIM), lambda i: (0, 0)),
          pl.BlockSpec((6, DIM), lambda i: (0, 0)),
          pl.BlockSpec((BLK // 2, BLK), lambda i: (0, 0)),
      ],
      out_specs=[
          pl.BlockSpec((BLK // 2, DIM), lambda i: (i, 0)),
          pl.BlockSpec((1, DIM), lambda i: (0, 0)),
      ],
      out_shape=[
          jax.ShapeDtypeStruct((n_pad // 2, DIM), jnp.float32),
          jax.ShapeDtypeStruct((1, DIM), jnp.float32),
      ],
  )


def _head_body(cs_ref, g_ref, bt_ref, w_ref, b_ref, o_ref):
  e = jax.nn.relu(g_ref[...] * (cs_ref[...] * INV_SQRT2) + bt_ref[...])
  acc = b_ref[...]
  for i in range(N_LAYERS):
    acc = acc + jnp.dot(e[i:i + 1, :], w_ref[i],
                        preferred_element_type=jnp.float32)
  o_ref[...] = acc


_head_call = pl.pallas_call(
    _head_body,
    in_specs=[
        pl.BlockSpec((N_LAYERS, DIM), lambda: (0, 0)),
        pl.BlockSpec((N_LAYERS, DIM), lambda: (0, 0)),
        pl.BlockSpec((N_LAYERS, DIM), lambda: (0, 0)),
        pl.BlockSpec((N_LAYERS, DIM, DIM), lambda: (0, 0, 0)),
        pl.BlockSpec((1, DIM), lambda: (0, 0)),
    ],
    out_specs=pl.BlockSpec((1, DIM), lambda: (0, 0)),
    out_shape=jax.ShapeDtypeStruct((1, DIM), jnp.float32),
)


def kernel(x, edge_index, Wstart, bstart, gstart, btstart, W1s, b1s, g1s,
           bt1s, W2s, b2s, gbns, btbns, gembd, btembd, Wlin, blin):
  f32 = jnp.float32
  xp = jnp.pad(x, ((0, N_PAD[0] - N_NODES), (0, 0)))
  src = jnp.concatenate(
      [edge_index[0], jnp.zeros((E_PAD - N_EDGES,), jnp.int32)])
  dst = jnp.concatenate(
      [edge_index[1],
       jnp.full((E_PAD - N_EDGES,), N_NODES, jnp.int32)])
  src3 = src.reshape(N_SUB, CHUNKS_PER_SUB, CHUNK)
  dst3 = dst.reshape(N_SUB, CHUNKS_PER_SUB, CHUNK)
  src32 = src.reshape(32, CHUNKS_FULL, CHUNK)
  dst32 = dst.reshape(32, CHUNKS_FULL, CHUNK)

  vstart = jnp.stack([bstart, gstart, btstart]).astype(f32)
  h = _make_start(N_PAD[0], N_NODES)(xp, Wstart, vstart)

  rr = lax.broadcasted_iota(jnp.int32, (BLK // 2, BLK), 0)
  cc = lax.broadcasted_iota(jnp.int32, (BLK // 2, BLK), 1)
  pmat = jnp.where(rr == cc // 2, INV_SQRT2, 0.0).astype(f32)

  css = []
  for i in range(N_LAYERS):
    if i == 0:
      agg = _make_sc_agg(N_PAD[i], i)(h.reshape(-1, HALF), src3, dst3)
    else:
      agg = _make_sc_agg_full(N_PAD[i], i)(h, src32, dst32)
    v6 = jnp.stack([b1s[i], g1s[i], bt1s[i], b2s[i], gbns[i], btbns[i]])
    h, cs = _make_mlp(N_PAD[i], N_REAL[i], i == 0)(h, agg, agg, W1s[i],
                                                   W2s[i], v6, pmat)
    css.append(cs)

  cs_all = jnp.concatenate(css, axis=0)
  w3 = jnp.pad(
      Wlin.reshape(N_LAYERS, DIM, N_CLASSES),
      ((0, 0), (0, 0), (0, DIM - N_CLASSES)))
  b3 = jnp.pad(blin, (0, DIM - N_CLASSES)).reshape(1, DIM)
  out = _head_call(cs_all, gembd, btembd, w3, b3)
  return out[:, :N_CLASSES]


# speedup vs baseline: 1.1911x; 1.1911x over previous
import functools
import math

import jax
import jax.numpy as jnp
from jax import lax
from jax.experimental import pallas as pl
from jax.experimental.pallas import tpu as pltpu
from jax.experimental.pallas import tpu_sc as plsc

N_NODES = 10000
DIM = 128
N_EDGES = 320000
N_LAYERS = 3
N_CLASSES = 10

BLK = 512
N_SUB = 16
CHUNK = 128
CHUNKS_PER_SUB = 160
E_PAD = N_SUB * CHUNKS_PER_SUB * CHUNK
HALF = DIM // 2

N_REAL = [N_NODES >> i for i in range(N_LAYERS)]
N_PAD = [-(-n // BLK) * BLK for n in N_REAL]

INV_SQRT2 = 1.0 / math.sqrt(2.0)


def _make_sc_agg(n_pad: int, shift: int):
  mesh = plsc.VectorSubcoreMesh(core_axis_name="c", subcore_axis_name="s")
  rows_per_sub = n_pad // N_SUB

  @functools.partial(
      pl.kernel,
      mesh=mesh,
      compiler_params=pltpu.CompilerParams(use_tc_tiling_on_sc=False),
      out_type=jax.ShapeDtypeStruct((2, n_pad, HALF), jnp.float32),
      scratch_types=[
          pltpu.VMEM((CHUNKS_PER_SUB, CHUNK), jnp.int32),
          pltpu.VMEM((CHUNKS_PER_SUB, CHUNK), jnp.int32),
          pltpu.VMEM((CHUNK, HALF), jnp.float32),
          pltpu.VMEM((CHUNK, HALF), jnp.float32),
          pltpu.VMEM((32, HALF), jnp.float32),
          pltpu.VMEM_SHARED((n_pad, HALF), jnp.float32),
          pltpu.SemaphoreType.DMA,
          pltpu.SemaphoreType.DMA,
      ],
  )
  def sc_agg(h2_hbm, src_hbm, dst_hbm, out_hbm, src_v, dst_v, rows0, rows1,
             zbuf, agg_sh, gs0, gs1):
    cid = lax.axis_index("c")
    sid = lax.axis_index("s")

    @pl.loop(0, 32)
    def _(r):
      @pl.loop(0, HALF, step=16)
      def _(t):
        zbuf[r, pl.ds(t, 16)] = jnp.zeros((16,), jnp.float32)

    base = sid * rows_per_sub

    @pl.loop(0, rows_per_sub, step=32)
    def _(r):
      pltpu.sync_copy(zbuf, agg_sh.at[pl.ds(base + r, 32)])

    pltpu.sync_copy(src_hbm.at[sid], src_v)
    pltpu.sync_copy(dst_hbm.at[sid], dst_v)

    @pl.loop(0, CHUNKS_PER_SUB)
    def _(j):
      for t in range(CHUNK // 16):
        sl = pl.ds(t * 16, 16)
        s = src_v[j, sl]
        if shift:
          s = lax.shift_right_logical(s, shift)
          dst_v[j, sl] = lax.shift_right_logical(dst_v[j, sl], shift)
        src_v[j, sl] = lax.shift_left(s, 1) | cid

    plsc.subcore_barrier()

    pltpu.async_copy(h2_hbm.at[src_v.at[0]], rows0, gs0)

    @pl.loop(0, CHUNKS_PER_SUB, step=2)
    def _(j):
      pltpu.async_copy(h2_hbm.at[src_v.at[j + 1]], rows1, gs1)
      pltpu.make_async_copy(h2_hbm.at[pl.ds(0, CHUNK)], rows0, gs0).wait()
      pltpu.sync_copy(rows0, agg_sh.at[dst_v.at[j]], add=True)

      @pl.when(j + 2 < CHUNKS_PER_SUB)
      def _():
        pltpu.async_copy(h2_hbm.at[src_v.at[j + 2]], rows0, gs0)

      pltpu.make_async_copy(h2_hbm.at[pl.ds(0, CHUNK)], rows1, gs1).wait()
      pltpu.sync_copy(rows1, agg_sh.at[dst_v.at[j + 1]], add=True)

    plsc.subcore_barrier()

    @pl.loop(0, rows_per_sub, step=32)
    def _(r):
      pltpu.sync_copy(agg_sh.at[pl.ds(base + r, 32)], zbuf)
      pltpu.sync_copy(zbuf, out_hbm.at[cid, pl.ds(base + r, 32)])

  return sc_agg


def _start_body(n_real, x_ref, w_ref, v_ref, h_ref):
  i = pl.program_id(0)
  z = jnp.dot(x_ref[...], w_ref[...], preferred_element_type=jnp.float32)
  h = jax.nn.relu(v_ref[1:2, :] * (z + v_ref[0:1, :]) + v_ref[2:3, :])
  row = i * BLK + lax.broadcasted_iota(jnp.int32, (BLK, 1), 0)
  h_ref[...] = jnp.where(row < n_real, h, 0.0)


def _make_start(n_pad, n_real):
  grid = n_pad // BLK
  return pl.pallas_call(
      functools.partial(_start_body, n_real),
      grid=(grid,),
      in_specs=[
          pl.BlockSpec((BLK, DIM), lambda i: (i, 0)),
          pl.BlockSpec((DIM, DIM), lambda i: (0, 0)),
          pl.BlockSpec((3, DIM), lambda i: (0, 0)),
      ],
      out_specs=pl.BlockSpec((BLK, DIM), lambda i: (i, 0)),
      out_shape=jax.ShapeDtypeStruct((n_pad, DIM), jnp.float32),
  )


def _mlp_body(n_real, grid, last, *refs):
  if last:
    (h_ref, a0_ref, a1_ref, w1_ref, w2_ref, v_ref, p_ref, cs01_ref, ge_ref,
     be_ref, w3_ref, b3_ref, pool_ref, cs_ref, o_ref) = refs
  else:
    (h_ref, a0_ref, a1_ref, w1_ref, w2_ref, v_ref, p_ref, pool_ref,
     cs_ref) = refs
  i = pl.program_id(0)
  agg = jnp.concatenate([a0_ref[0], a1_ref[0]], axis=-1)
  g = h_ref[...] + agg
  z = jnp.dot(g, w1_ref[...], preferred_element_type=jnp.float32)
  z = jax.nn.relu(v_ref[1:2, :] * (z + v_ref[0:1, :]) + v_ref[2:3, :])
  u = jnp.dot(z, w2_ref[...], preferred_element_type=jnp.float32)
  u = jax.nn.relu(v_ref[4:5, :] * (u + v_ref[3:4, :]) + v_ref[5:6, :])
  row = i * BLK + lax.broadcasted_iota(jnp.int32, (BLK, 1), 0)
  u = jnp.where(row < n_real, u, 0.0)
  pool_ref[...] = jnp.dot(p_ref[...], u, preferred_element_type=jnp.float32)
  cs = jnp.sum(u, axis=0, keepdims=True)

  @pl.when(i == 0)
  def _():
    cs_ref[...] = cs

  @pl.when(i > 0)
  def _():
    cs_ref[...] += cs

  if last:
    @pl.when(i == grid - 1)
    def _():
      e0 = jax.nn.relu(ge_ref[0:1, :] * (cs01_ref[0:1, :] * INV_SQRT2)
                       + be_ref[0:1, :])
      e1 = jax.nn.relu(ge_ref[1:2, :] * (cs01_ref[1:2, :] * INV_SQRT2)
                       + be_ref[1:2, :])
      e2 = jax.nn.relu(ge_ref[2:3, :] * (cs_ref[...] * INV_SQRT2)
                       + be_ref[2:3, :])
      acc = b3_ref[...]
      for k, e in enumerate((e0, e1, e2)):
        acc = acc + jnp.dot(e, w3_ref[k], preferred_element_type=jnp.float32)
      o_ref[...] = acc


def _make_mlp(n_pad, n_real, last):
  grid = n_pad // BLK
  in_specs = [
      pl.BlockSpec((BLK, DIM), lambda i: (i, 0)),
      pl.BlockSpec((1, BLK, HALF), lambda i: (0, i, 0)),
      pl.BlockSpec((1, BLK, HALF), lambda i: (1, i, 0)),
      pl.BlockSpec((DIM, DIM), lambda i: (0, 0)),
      pl.BlockSpec((DIM, DIM), lambda i: (0, 0)),
      pl.BlockSpec((6, DIM), lambda i: (0, 0)),
      pl.BlockSpec((BLK // 2, BLK), lambda i: (0, 0)),
  ]
  out_specs = [
      pl.BlockSpec((BLK // 2, DIM), lambda i: (i, 0)),
      pl.BlockSpec((1, DIM), lambda i: (0, 0)),
  ]
  out_shape = [
      jax.ShapeDtypeStruct((n_pad // 2, DIM), jnp.float32),
      jax.ShapeDtypeStruct((1, DIM), jnp.float32),
  ]
  if last:
    in_specs += [
        pl.BlockSpec((2, DIM), lambda i: (0, 0)),
        pl.BlockSpec((N_LAYERS, DIM), lambda i: (0, 0)),
        pl.BlockSpec((N_LAYERS, DIM), lambda i: (0, 0)),
        pl.BlockSpec((N_LAYERS, DIM, DIM), lambda i: (0, 0, 0)),
        pl.BlockSpec((1, DIM), lambda i: (0, 0)),
    ]
    out_specs.append(pl.BlockSpec((1, DIM), lambda i: (0, 0)))
    out_shape.append(jax.ShapeDtypeStruct((1, DIM), jnp.float32))
  return pl.pallas_call(
      functools.partial(_mlp_body, n_real, grid, last),
      grid=(grid,),
      in_specs=in_specs,
      out_specs=out_specs,
      out_shape=out_shape,
  )


def kernel(x, edge_index, Wstart, bstart, gstart, btstart, W1s, b1s, g1s,
           bt1s, W2s, b2s, gbns, btbns, gembd, btembd, Wlin, blin):
  f32 = jnp.float32
  src = jnp.concatenate(
      [edge_index[0], jnp.zeros((E_PAD - N_EDGES,), jnp.int32)])
  dst = jnp.concatenate(
      [edge_index[1],
       jnp.full((E_PAD - N_EDGES,), N_NODES, jnp.int32)])
  src3 = src.reshape(N_SUB, CHUNKS_PER_SUB, CHUNK)
  dst3 = dst.reshape(N_SUB, CHUNKS_PER_SUB, CHUNK)

  vstart = jnp.stack([bstart, gstart, btstart]).astype(f32)
  h = _make_start(N_PAD[0], N_NODES)(x, Wstart, vstart)

  rr = lax.broadcasted_iota(jnp.int32, (BLK // 2, BLK), 0)
  cc = lax.broadcasted_iota(jnp.int32, (BLK // 2, BLK), 1)
  pmat = jnp.where(rr == cc // 2, INV_SQRT2, 0.0).astype(f32)

  w3 = jnp.pad(
      Wlin.reshape(N_LAYERS, DIM, N_CLASSES),
      ((0, 0), (0, 0), (0, DIM - N_CLASSES)))
  b3 = jnp.pad(blin, (0, DIM - N_CLASSES)).reshape(1, DIM)

  css = []
  for i in range(N_LAYERS):
    agg = _make_sc_agg(N_PAD[i], i)(h.reshape(-1, HALF), src3, dst3)
    v6 = jnp.stack([b1s[i], g1s[i], bt1s[i], b2s[i], gbns[i], btbns[i]])
    mlp = _make_mlp(N_PAD[i], N_REAL[i], i == N_LAYERS - 1)
    if i < N_LAYERS - 1:
      h, cs = mlp(h, agg, agg, W1s[i], W2s[i], v6, pmat)
      css.append(cs)
    else:
      cs01 = jnp.concatenate(css, axis=0)
      _, _, out = mlp(h, agg, agg, W1s[i], W2s[i], v6, pmat, cs01, gembd,
                      btembd, w3, b3)
  return out[:, :N_CLASSES]


# speedup vs baseline: 3.5996x; 3.0220x over previous
import functools
import math

import jax
import jax.numpy as jnp
from jax import lax
from jax.experimental import pallas as pl
from jax.experimental.pallas import tpu as pltpu
from jax.experimental.pallas import tpu_sc as plsc

N_NODES = 10000
DIM = 128
N_EDGES = 320000
N_LAYERS = 3
N_CLASSES = 10

BLK = 512
N_SUB = 16
CHUNK = 128
CHUNKS_PER_SUB = 160
E_PAD = N_SUB * CHUNKS_PER_SUB * CHUNK
HALF = DIM // 2

N_REAL = [N_NODES >> i for i in range(N_LAYERS)]
N_PAD = [-(-n // BLK) * BLK for n in N_REAL]

INV_SQRT2 = 1.0 / math.sqrt(2.0)


def _make_sc_agg(n_pad: int, shift: int):
  mesh = plsc.VectorSubcoreMesh(core_axis_name="c", subcore_axis_name="s")
  rows_per_sub = n_pad // N_SUB

  @functools.partial(
      pl.kernel,
      mesh=mesh,
      compiler_params=pltpu.CompilerParams(use_tc_tiling_on_sc=False),
      out_type=jax.ShapeDtypeStruct((2, n_pad, HALF), jnp.float32),
      scratch_types=[
          pltpu.VMEM((CHUNKS_PER_SUB, CHUNK), jnp.int32),
          pltpu.VMEM((CHUNKS_PER_SUB, CHUNK), jnp.int32),
          pltpu.VMEM((CHUNK, HALF), jnp.float32),
          pltpu.VMEM((CHUNK, HALF), jnp.float32),
          pltpu.VMEM((32, HALF), jnp.float32),
          pltpu.VMEM_SHARED((n_pad, HALF), jnp.float32),
          pltpu.SemaphoreType.DMA,
          pltpu.SemaphoreType.DMA,
      ],
  )
  def sc_agg(h2_hbm, src_hbm, dst_hbm, out_hbm, src_v, dst_v, rows0, rows1,
             zbuf, agg_sh, gs0, gs1):
    cid = lax.axis_index("c")
    sid = lax.axis_index("s")

    @pl.loop(0, 32)
    def _(r):
      @pl.loop(0, HALF, step=16)
      def _(t):
        zbuf[r, pl.ds(t, 16)] = jnp.zeros((16,), jnp.float32)

    base = sid * rows_per_sub

    @pl.loop(0, rows_per_sub, step=32)
    def _(r):
      pltpu.sync_copy(zbuf, agg_sh.at[pl.ds(base + r, 32)])

    pltpu.sync_copy(src_hbm.at[sid], src_v)
    pltpu.sync_copy(dst_hbm.at[sid], dst_v)

    @pl.loop(0, CHUNKS_PER_SUB)
    def _(j):
      for t in range(CHUNK // 16):
        sl = pl.ds(t * 16, 16)
        s = src_v[j, sl]
        if shift:
          s = lax.shift_right_logical(s, shift)
          dst_v[j, sl] = lax.shift_right_logical(dst_v[j, sl], shift)
        src_v[j, sl] = lax.shift_left(s, 1) | cid

    plsc.subcore_barrier()

    pltpu.async_copy(h2_hbm.at[src_v.at[0]], rows0, gs0)

    @pl.loop(0, CHUNKS_PER_SUB, step=2)
    def _(j):
      pltpu.async_copy(h2_hbm.at[src_v.at[j + 1]], rows1, gs1)
      pltpu.make_async_copy(h2_hbm.at[pl.ds(0, CHUNK)], rows0, gs0).wait()
      pltpu.sync_copy(rows0, agg_sh.at[dst_v.at[j]], add=True)

      @pl.when(j + 2 < CHUNKS_PER_SUB)
      def _():
        pltpu.async_copy(h2_hbm.at[src_v.at[j + 2]], rows0, gs0)

      pltpu.make_async_copy(h2_hbm.at[pl.ds(0, CHUNK)], rows1, gs1).wait()
      pltpu.sync_copy(rows1, agg_sh.at[dst_v.at[j + 1]], add=True)

    plsc.subcore_barrier()

    @pl.loop(0, rows_per_sub, step=32)
    def _(r):
      pltpu.sync_copy(agg_sh.at[pl.ds(base + r, 32)], zbuf)
      pltpu.sync_copy(zbuf, out_hbm.at[cid, pl.ds(base + r, 32)])

  return sc_agg


CHUNKS_FULL = E_PAD // (32 * CHUNK)


def _make_sc_agg_full(n_pad: int, shift: int):
  mesh = plsc.VectorSubcoreMesh(core_axis_name="c", subcore_axis_name="s")
  rows_per_sub = n_pad // N_SUB

  @functools.partial(
      pl.kernel,
      mesh=mesh,
      out_type=jax.ShapeDtypeStruct((2, n_pad, DIM), jnp.float32),
      scratch_types=[
          pltpu.VMEM((CHUNKS_FULL, CHUNK), jnp.int32),
          pltpu.VMEM((CHUNKS_FULL, CHUNK), jnp.int32),
          pltpu.VMEM((CHUNK, DIM), jnp.float32),
          pltpu.VMEM((CHUNK, DIM), jnp.float32),
          pltpu.VMEM((32, DIM), jnp.float32),
          pltpu.VMEM_SHARED((n_pad, DIM), jnp.float32),
          pltpu.SemaphoreType.DMA,
          pltpu.SemaphoreType.DMA,
      ],
  )
  def sc_agg(h_hbm, src_hbm, dst_hbm, out_hbm, src_v, dst_v, rows0, rows1,
             zbuf, agg_sh, gs0, gs1):
    cid = lax.axis_index("c")
    sid = lax.axis_index("s")
    wid = sid * 2 + cid

    @pl.loop(0, 32)
    def _(r):
      @pl.loop(0, DIM, step=16)
      def _(t):
        zbuf[r, pl.ds(t, 16)] = jnp.zeros((16,), jnp.float32)

    base = sid * rows_per_sub

    @pl.loop(0, rows_per_sub, step=32)
    def _(r):
      pltpu.sync_copy(zbuf, agg_sh.at[pl.ds(base + r, 32)])

    pltpu.sync_copy(src_hbm.at[wid], src_v)
    pltpu.sync_copy(dst_hbm.at[wid], dst_v)
    if shift:
      @pl.loop(0, CHUNKS_FULL)
      def _(j):
        for t in range(CHUNK // 16):
          sl = pl.ds(t * 16, 16)
          src_v[j, sl] = lax.shift_right_logical(src_v[j, sl], shift)
          dst_v[j, sl] = lax.shift_right_logical(dst_v[j, sl], shift)

    plsc.subcore_barrier()

    pltpu.async_copy(h_hbm.at[src_v.at[0]], rows0, gs0)

    @pl.loop(0, CHUNKS_FULL, step=2)
    def _(j):
      pltpu.async_copy(h_hbm.at[src_v.at[j + 1]], rows1, gs1)
      pltpu.make_async_copy(h_hbm.at[pl.ds(0, CHUNK)], rows0, gs0).wait()
      pltpu.sync_copy(rows0, agg_sh.at[dst_v.at[j]], add=True)

      @pl.when(j + 2 < CHUNKS_FULL)
      def _():
        pltpu.async_copy(h_hbm.at[src_v.at[j + 2]], rows0, gs0)

      pltpu.make_async_copy(h_hbm.at[pl.ds(0, CHUNK)], rows1, gs1).wait()
      pltpu.sync_copy(rows1, agg_sh.at[dst_v.at[j + 1]], add=True)

    plsc.subcore_barrier()

    @pl.loop(0, rows_per_sub, step=32)
    def _(r):
      pltpu.sync_copy(agg_sh.at[pl.ds(base + r, 32)], zbuf)
      pltpu.sync_copy(zbuf, out_hbm.at[cid, pl.ds(base + r, 32)])

  return sc_agg


def _start_body(n_real, x_ref, w_ref, v_ref, h_ref):
  i = pl.program_id(0)
  z = jnp.dot(x_ref[...], w_ref[...], preferred_element_type=jnp.float32)
  h = jax.nn.relu(v_ref[1:2, :] * (z + v_ref[0:1, :]) + v_ref[2:3, :])
  row = i * BLK + lax.broadcasted_iota(jnp.int32, (BLK, 1), 0)
  h_ref[...] = jnp.where(row < n_real, h, 0.0)


def _make_start(n_pad, n_real):
  grid = n_pad // BLK
  return pl.pallas_call(
      functools.partial(_start_body, n_real),
      grid=(grid,),
      in_specs=[
          pl.BlockSpec((BLK, DIM), lambda i: (i, 0)),
          pl.BlockSpec((DIM, DIM), lambda i: (0, 0)),
          pl.BlockSpec((3, DIM), lambda i: (0, 0)),
      ],
      out_specs=pl.BlockSpec((BLK, DIM), lambda i: (i, 0)),
      out_shape=jax.ShapeDtypeStruct((n_pad, DIM), jnp.float32),
  )


def _mlp_body(n_real, grid, col_split, last, *refs):
  if last:
    (h_ref, a0_ref, a1_ref, w1_ref, w2_ref, v_ref, p_ref, cs01_ref, ge_ref,
     be_ref, w3_ref, b3_ref, pool_ref, cs_ref, o_ref) = refs
  else:
    (h_ref, a0_ref, a1_ref, w1_ref, w2_ref, v_ref, p_ref, pool_ref,
     cs_ref) = refs
  i = pl.program_id(0)
  if col_split:
    agg = jnp.concatenate([a0_ref[0], a1_ref[0]], axis=-1)
  else:
    agg = a0_ref[0] + a1_ref[0]
  g = h_ref[...] + agg
  z = jnp.dot(g, w1_ref[...], preferred_element_type=jnp.float32)
  z = jax.nn.relu(v_ref[1:2, :] * (z + v_ref[0:1, :]) + v_ref[2:3, :])
  u = jnp.dot(z, w2_ref[...], preferred_element_type=jnp.float32)
  u = jax.nn.relu(v_ref[4:5, :] * (u + v_ref[3:4, :]) + v_ref[5:6, :])
  row = i * BLK + lax.broadcasted_iota(jnp.int32, (BLK, 1), 0)
  u = jnp.where(row < n_real, u, 0.0)
  pool_ref[...] = jnp.dot(p_ref[...], u, preferred_element_type=jnp.float32)
  cs = jnp.sum(u, axis=0, keepdims=True)

  @pl.when(i == 0)
  def _():
    cs_ref[...] = cs

  @pl.when(i > 0)
  def _():
    cs_ref[...] += cs

  if last:
    @pl.when(i == grid - 1)
    def _():
      e0 = jax.nn.relu(ge_ref[0:1, :] * (cs01_ref[0:1, :] * INV_SQRT2)
                       + be_ref[0:1, :])
      e1 = jax.nn.relu(ge_ref[1:2, :] * (cs01_ref[1:2, :] * INV_SQRT2)
                       + be_ref[1:2, :])
      e2 = jax.nn.relu(ge_ref[2:3, :] * (cs_ref[...] * INV_SQRT2)
                       + be_ref[2:3, :])
      acc = b3_ref[...]
      for k, e in enumerate((e0, e1, e2)):
        acc = acc + jnp.dot(e, w3_ref[k], preferred_element_type=jnp.float32)
      o_ref[...] = acc


def _make_mlp(n_pad, n_real, col_split, last):
  grid = n_pad // BLK
  aw = HALF if col_split else DIM
  in_specs = [
      pl.BlockSpec((BLK, DIM), lambda i: (i, 0)),
      pl.BlockSpec((1, BLK, aw), lambda i: (0, i, 0)),
      pl.BlockSpec((1, BLK, aw), lambda i: (1, i, 0)),
      pl.BlockSpec((DIM, DIM), lambda i: (0, 0)),
      pl.BlockSpec((DIM, DIM), lambda i: (0, 0)),
      pl.BlockSpec((6, DIM), lambda i: (0, 0)),
      pl.BlockSpec((BLK // 2, BLK), lambda i: (0, 0)),
  ]
  out_specs = [
      pl.BlockSpec((BLK // 2, DIM), lambda i: (i, 0)),
      pl.BlockSpec((1, DIM), lambda i: (0, 0)),
  ]
  out_shape = [
      jax.ShapeDtypeStruct((n_pad // 2, DIM), jnp.float32),
      jax.ShapeDtypeStruct((1, DIM), jnp.float32),
  ]
  if last:
    in_specs += [
        pl.BlockSpec((2, DIM), lambda i: (0, 0)),
        pl.BlockSpec((N_LAYERS, DIM), lambda i: (0, 0)),
        pl.BlockSpec((N_LAYERS, DIM), lambda i: (0, 0)),
        pl.BlockSpec((N_LAYERS, DIM, DIM), lambda i: (0, 0, 0)),
        pl.BlockSpec((1, DIM), lambda i: (0, 0)),
    ]
    out_specs.append(pl.BlockSpec((1, DIM), lambda i: (0, 0)))
    out_shape.append(jax.ShapeDtypeStruct((1, DIM), jnp.float32))
  return pl.pallas_call(
      functools.partial(_mlp_body, n_real, grid, col_split, last),
      grid=(grid,),
      in_specs=in_specs,
      out_specs=out_specs,
      out_shape=out_shape,
  )


def kernel(x, edge_index, Wstart, bstart, gstart, btstart, W1s, b1s, g1s,
           bt1s, W2s, b2s, gbns, btbns, gembd, btembd, Wlin, blin):
  f32 = jnp.float32
  src = jnp.concatenate(
      [edge_index[0], jnp.zeros((E_PAD - N_EDGES,), jnp.int32)])
  trash = N_NODES + jnp.arange(E_PAD - N_EDGES, dtype=jnp.int32) % (
      N_PAD[0] - N_NODES)
  dst = jnp.concatenate([edge_index[1], trash])
  src3 = src.reshape(N_SUB, CHUNKS_PER_SUB, CHUNK)
  dst3 = dst.reshape(N_SUB, CHUNKS_PER_SUB, CHUNK)
  src32 = src.reshape(32, CHUNKS_FULL, CHUNK)
  dst32 = dst.reshape(32, CHUNKS_FULL, CHUNK)

  vstart = jnp.stack([bstart, gstart, btstart]).astype(f32)
  h = _make_start(N_PAD[0], N_NODES)(x, Wstart, vstart)

  rr = lax.broadcasted_iota(jnp.int32, (BLK // 2, BLK), 0)
  cc = lax.broadcasted_iota(jnp.int32, (BLK // 2, BLK), 1)
  pmat = jnp.where(rr == cc // 2, INV_SQRT2, 0.0).astype(f32)

  w3 = jnp.pad(
      Wlin.reshape(N_LAYERS, DIM, N_CLASSES),
      ((0, 0), (0, 0), (0, DIM - N_CLASSES)))
  b3 = jnp.pad(blin, (0, DIM - N_CLASSES)).reshape(1, DIM)

  css = []
  for i in range(N_LAYERS):
    col_split = i == 0
    if col_split:
      agg = _make_sc_agg(N_PAD[i], i)(h.reshape(-1, HALF), src3, dst3)
    else:
      agg = _make_sc_agg_full(N_PAD[i], i)(h, src32, dst32)
    v6 = jnp.stack([b1s[i], g1s[i], bt1s[i], b2s[i], gbns[i], btbns[i]])
    mlp = _make_mlp(N_PAD[i], N_REAL[i], col_split, i == N_LAYERS - 1)
    if i < N_LAYERS - 1:
      h, cs = mlp(h, agg, agg, W1s[i], W2s[i], v6, pmat)
      css.append(cs)
    else:
      cs01 = jnp.concatenate(css, axis=0)
      _, _, out = mlp(h, agg, agg, W1s[i], W2s[i], v6, pmat, cs01, gembd,
                      btembd, w3, b3)
  return out[:, :N_CLASSES]


# speedup vs baseline: 3.5998x; 1.0000x over previous
import functools
import math

import jax
import jax.numpy as jnp
from jax import lax
from jax.experimental import pallas as pl
from jax.experimental.pallas import tpu as pltpu
from jax.experimental.pallas import tpu_sc as plsc

N_NODES = 10000
DIM = 128
N_EDGES = 320000
N_LAYERS = 3
N_CLASSES = 10

BLK = 512
N_SUB = 16
CHUNK = 128
CHUNKS_PER_SUB = 160
E_PAD = N_SUB * CHUNKS_PER_SUB * CHUNK
HALF = DIM // 2

N_REAL = [N_NODES >> i for i in range(N_LAYERS)]
N_PAD = [-(-n // BLK) * BLK for n in N_REAL]

INV_SQRT2 = 1.0 / math.sqrt(2.0)


def _make_sc_agg(n_pad: int, shift: int):
  mesh = plsc.VectorSubcoreMesh(core_axis_name="c", subcore_axis_name="s")
  rows_per_sub = n_pad // N_SUB

  @functools.partial(
      pl.kernel,
      mesh=mesh,
      compiler_params=pltpu.CompilerParams(use_tc_tiling_on_sc=False),
      out_type=jax.ShapeDtypeStruct((2, n_pad, HALF), jnp.float32),
      scratch_types=[
          pltpu.VMEM((CHUNKS_PER_SUB, CHUNK), jnp.int32),
          pltpu.VMEM((CHUNKS_PER_SUB, CHUNK), jnp.int32),
          pltpu.VMEM((CHUNK, HALF), jnp.float32),
          pltpu.VMEM((CHUNK, HALF), jnp.float32),
          pltpu.VMEM((32, HALF), jnp.float32),
          pltpu.VMEM_SHARED((n_pad, HALF), jnp.float32),
          pltpu.SemaphoreType.DMA,
          pltpu.SemaphoreType.DMA,
      ],
  )
  def sc_agg(h2_hbm, src_hbm, dst_hbm, out_hbm, src_v, dst_v, rows0, rows1,
             zbuf, agg_sh, gs0, gs1):
    cid = lax.axis_index("c")
    sid = lax.axis_index("s")

    @pl.loop(0, 32)
    def _(r):
      @pl.loop(0, HALF, step=16)
      def _(t):
        zbuf[r, pl.ds(t, 16)] = jnp.zeros((16,), jnp.float32)

    base = sid * rows_per_sub

    @pl.loop(0, rows_per_sub, step=32)
    def _(r):
      pltpu.sync_copy(zbuf, agg_sh.at[pl.ds(base + r, 32)])

    pltpu.sync_copy(src_hbm.at[sid], src_v)
    pltpu.sync_copy(dst_hbm.at[sid], dst_v)

    @pl.loop(0, CHUNKS_PER_SUB)
    def _(j):
      for t in range(CHUNK // 16):
        sl = pl.ds(t * 16, 16)
        s = src_v[j, sl]
        if shift:
          s = lax.shift_right_logical(s, shift)
          dst_v[j, sl] = lax.shift_right_logical(dst_v[j, sl], shift)
        src_v[j, sl] = lax.shift_left(s, 1) | cid

    plsc.subcore_barrier()

    pltpu.async_copy(h2_hbm.at[src_v.at[0]], rows0, gs0)

    @pl.loop(0, CHUNKS_PER_SUB, step=2)
    def _(j):
      pltpu.async_copy(h2_hbm.at[src_v.at[j + 1]], rows1, gs1)
      pltpu.make_async_copy(h2_hbm.at[pl.ds(0, CHUNK)], rows0, gs0).wait()
      pltpu.sync_copy(rows0, agg_sh.at[dst_v.at[j]], add=True)

      @pl.when(j + 2 < CHUNKS_PER_SUB)
      def _():
        pltpu.async_copy(h2_hbm.at[src_v.at[j + 2]], rows0, gs0)

      pltpu.make_async_copy(h2_hbm.at[pl.ds(0, CHUNK)], rows1, gs1).wait()
      pltpu.sync_copy(rows1, agg_sh.at[dst_v.at[j + 1]], add=True)

    plsc.subcore_barrier()

    @pl.loop(0, rows_per_sub, step=32)
    def _(r):
      pltpu.sync_copy(agg_sh.at[pl.ds(base + r, 32)], zbuf)
      pltpu.sync_copy(zbuf, out_hbm.at[cid, pl.ds(base + r, 32)])

  return sc_agg


CHUNKS_FULL = E_PAD // (32 * CHUNK)


def _make_sc_agg_full(n_pad: int, shift: int):
  mesh = plsc.VectorSubcoreMesh(core_axis_name="c", subcore_axis_name="s")
  rows_per_sub = n_pad // N_SUB

  @functools.partial(
      pl.kernel,
      mesh=mesh,
      out_type=jax.ShapeDtypeStruct((2, n_pad, DIM), jnp.float32),
      scratch_types=[
          pltpu.VMEM((CHUNKS_FULL, CHUNK), jnp.int32),
          pltpu.VMEM((CHUNKS_FULL, CHUNK), jnp.int32),
          pltpu.VMEM((CHUNK, DIM), jnp.float32),
          pltpu.VMEM((CHUNK, DIM), jnp.float32),
          pltpu.VMEM((32, DIM), jnp.float32),
          pltpu.VMEM_SHARED((n_pad, DIM), jnp.float32),
          pltpu.SemaphoreType.DMA,
          pltpu.SemaphoreType.DMA,
      ],
  )
  def sc_agg(h_hbm, src_hbm, dst_hbm, out_hbm, src_v, dst_v, rows0, rows1,
             zbuf, agg_sh, gs0, gs1):
    cid = lax.axis_index("c")
    sid = lax.axis_index("s")
    wid = sid * 2 + cid

    @pl.loop(0, 32)
    def _(r):
      @pl.loop(0, DIM, step=16)
      def _(t):
        zbuf[r, pl.ds(t, 16)] = jnp.zeros((16,), jnp.float32)

    base = sid * rows_per_sub

    @pl.loop(0, rows_per_sub, step=32)
    def _(r):
      pltpu.sync_copy(zbuf, agg_sh.at[pl.ds(base + r, 32)])

    pltpu.sync_copy(src_hbm.at[wid], src_v)
    pltpu.sync_copy(dst_hbm.at[wid], dst_v)
    if shift:
      @pl.loop(0, CHUNKS_FULL)
      def _(j):
        for t in range(CHUNK // 16):
          sl = pl.ds(t * 16, 16)
          src_v[j, sl] = lax.shift_right_logical(src_v[j, sl], shift)
          dst_v[j, sl] = lax.shift_right_logical(dst_v[j, sl], shift)

    plsc.subcore_barrier()

    pltpu.async_copy(h_hbm.at[src_v.at[0]], rows0, gs0)

    @pl.loop(0, CHUNKS_FULL, step=2)
    def _(j):
      pltpu.async_copy(h_hbm.at[src_v.at[j + 1]], rows1, gs1)
      pltpu.make_async_copy(h_hbm.at[pl.ds(0, CHUNK)], rows0, gs0).wait()
      pltpu.sync_copy(rows0, agg_sh.at[dst_v.at[j]], add=True)

      @pl.when(j + 2 < CHUNKS_FULL)
      def _():
        pltpu.async_copy(h_hbm.at[src_v.at[j + 2]], rows0, gs0)

      pltpu.make_async_copy(h_hbm.at[pl.ds(0, CHUNK)], rows1, gs1).wait()
      pltpu.sync_copy(rows1, agg_sh.at[dst_v.at[j + 1]], add=True)

    plsc.subcore_barrier()

    @pl.loop(0, rows_per_sub, step=32)
    def _(r):
      pltpu.sync_copy(agg_sh.at[pl.ds(base + r, 32)], zbuf)
      pltpu.sync_copy(zbuf, out_hbm.at[cid, pl.ds(base + r, 32)])

  return sc_agg


def _start_body(n_real, x_ref, w_ref, v_ref, h_ref):
  i = pl.program_id(0)
  z = jnp.dot(x_ref[...], w_ref[...], preferred_element_type=jnp.float32)
  h = jax.nn.relu(v_ref[1:2, :] * (z + v_ref[0:1, :]) + v_ref[2:3, :])
  row = i * BLK + lax.broadcasted_iota(jnp.int32, (BLK, 1), 0)
  h_ref[...] = jnp.where(row < n_real, h, 0.0)


def _make_start(n_pad, n_real):
  grid = n_pad // BLK
  return pl.pallas_call(
      functools.partial(_start_body, n_real),
      grid=(grid,),
      in_specs=[
          pl.BlockSpec((BLK, DIM), lambda i: (i, 0)),
          pl.BlockSpec((DIM, DIM), lambda i: (0, 0)),
          pl.BlockSpec((3, DIM), lambda i: (0, 0)),
      ],
      out_specs=pl.BlockSpec((BLK, DIM), lambda i: (i, 0)),
      out_shape=jax.ShapeDtypeStruct((n_pad, DIM), jnp.float32),
  )


def _mlp_body(n_real, grid, col_split, last, *refs):
  if last:
    (h_ref, a0_ref, a1_ref, w1_ref, w2_ref, v_ref, p_ref, cs01_ref, ge_ref,
     be_ref, w3_ref, b3_ref, pool_ref, cs_ref, o_ref) = refs
  else:
    (h_ref, a0_ref, a1_ref, w1_ref, w2_ref, v_ref, p_ref, pool_ref,
     cs_ref) = refs
  i = pl.program_id(0)
  if col_split:
    agg = jnp.concatenate([a0_ref[0], a1_ref[0]], axis=-1)
  else:
    agg = a0_ref[0] + a1_ref[0]
  g = h_ref[...] + agg
  z = jnp.dot(g, w1_ref[...], preferred_element_type=jnp.float32)
  z = jax.nn.relu(v_ref[1:2, :] * (z + v_ref[0:1, :]) + v_ref[2:3, :])
  u = jnp.dot(z, w2_ref[...], preferred_element_type=jnp.float32)
  u = jax.nn.relu(v_ref[4:5, :] * (u + v_ref[3:4, :]) + v_ref[5:6, :])
  row = i * BLK + lax.broadcasted_iota(jnp.int32, (BLK, 1), 0)
  u = jnp.where(row < n_real, u, 0.0)
  pool_ref[...] = jnp.dot(p_ref[...], u, preferred_element_type=jnp.float32)
  cs = jnp.sum(u, axis=0, keepdims=True)

  @pl.when(i == 0)
  def _():
    cs_ref[...] = cs

  @pl.when(i > 0)
  def _():
    cs_ref[...] += cs

  if last:
    @pl.when(i == grid - 1)
    def _():
      e0 = jax.nn.relu(ge_ref[0:1, :] * (cs01_ref[0:1, :] * INV_SQRT2)
                       + be_ref[0:1, :])
      e1 = jax.nn.relu(ge_ref[1:2, :] * (cs01_ref[1:2, :] * INV_SQRT2)
                       + be_ref[1:2, :])
      e2 = jax.nn.relu(ge_ref[2:3, :] * (cs_ref[...] * INV_SQRT2)
                       + be_ref[2:3, :])
      acc = b3_ref[...]
      for k, e in enumerate((e0, e1, e2)):
        acc = acc + jnp.dot(e, w3_ref[k], preferred_element_type=jnp.float32)
      o_ref[...] = acc


def _make_mlp(n_pad, n_real, col_split, last):
  grid = n_pad // BLK
  aw = HALF if col_split else DIM
  in_specs = [
      pl.BlockSpec((BLK, DIM), lambda i: (i, 0)),
      pl.BlockSpec((1, BLK, aw), lambda i: (0, i, 0)),
      pl.BlockSpec((1, BLK, aw), lambda i: (1, i, 0)),
      pl.BlockSpec((DIM, DIM), lambda i: (0, 0)),
      pl.BlockSpec((DIM, DIM), lambda i: (0, 0)),
      pl.BlockSpec((6, DIM), lambda i: (0, 0)),
      pl.BlockSpec((BLK // 2, BLK), lambda i: (0, 0)),
  ]
  out_specs = [
      pl.BlockSpec((BLK // 2, DIM), lambda i: (i, 0)),
      pl.BlockSpec((1, DIM), lambda i: (0, 0)),
  ]
  out_shape = [
      jax.ShapeDtypeStruct((n_pad // 2, DIM), jnp.float32),
      jax.ShapeDtypeStruct((1, DIM), jnp.float32),
  ]
  if last:
    in_specs += [
        pl.BlockSpec((2, DIM), lambda i: (0, 0)),
        pl.BlockSpec((N_LAYERS, DIM), lambda i: (0, 0)),
        pl.BlockSpec((N_LAYERS, DIM), lambda i: (0, 0)),
        pl.BlockSpec((N_LAYERS, DIM, DIM), lambda i: (0, 0, 0)),
        pl.BlockSpec((1, DIM), lambda i: (0, 0)),
    ]
    out_specs.append(pl.BlockSpec((1, DIM), lambda i: (0, 0)))
    out_shape.append(jax.ShapeDtypeStruct((1, DIM), jnp.float32))
  return pl.pallas_call(
      functools.partial(_mlp_body, n_real, grid, col_split, last),
      grid=(grid,),
      in_specs=in_specs,
      out_specs=out_specs,
      out_shape=out_shape,
  )


def kernel(x, edge_index, Wstart, bstart, gstart, btstart, W1s, b1s, g1s,
           bt1s, W2s, b2s, gbns, btbns, gembd, btembd, Wlin, blin):
  f32 = jnp.float32
  pad_ids = jnp.arange(E_PAD - N_EDGES, dtype=jnp.int32)
  src = jnp.concatenate([edge_index[0], pad_ids % N_NODES])
  dst = jnp.concatenate(
      [edge_index[1], N_NODES + pad_ids % (N_PAD[0] - N_NODES)])
  src3 = src.reshape(N_SUB, CHUNKS_PER_SUB, CHUNK)
  dst3 = dst.reshape(N_SUB, CHUNKS_PER_SUB, CHUNK)
  src32 = src.reshape(32, CHUNKS_FULL, CHUNK)
  dst32 = dst.reshape(32, CHUNKS_FULL, CHUNK)

  vstart = jnp.stack([bstart, gstart, btstart]).astype(f32)
  h = _make_start(N_PAD[0], N_NODES)(x, Wstart, vstart)

  rr = lax.broadcasted_iota(jnp.int32, (BLK // 2, BLK), 0)
  cc = lax.broadcasted_iota(jnp.int32, (BLK // 2, BLK), 1)
  pmat = jnp.where(rr == cc // 2, INV_SQRT2, 0.0).astype(f32)

  w3 = jnp.pad(
      Wlin.reshape(N_LAYERS, DIM, N_CLASSES),
      ((0, 0), (0, 0), (0, DIM - N_CLASSES)))
  b3 = jnp.pad(blin, (0, DIM - N_CLASSES)).reshape(1, DIM)

  css = []
  for i in range(N_LAYERS):
    col_split = i == 0
    if col_split:
      agg = _make_sc_agg(N_PAD[i], i)(h.reshape(-1, HALF), src3, dst3)
    else:
      agg = _make_sc_agg_full(N_PAD[i], i)(h, src32, dst32)
    v6 = jnp.stack([b1s[i], g1s[i], bt1s[i], b2s[i], gbns[i], btbns[i]])
    mlp = _make_mlp(N_PAD[i], N_REAL[i], col_split, i == N_LAYERS - 1)
    if i < N_LAYERS - 1:
      h, cs = mlp(h, agg, agg, W1s[i], W2s[i], v6, pmat)
      css.append(cs)
    else:
      cs01 = jnp.concatenate(css, axis=0)
      _, _, out = mlp(h, agg, agg, W1s[i], W2s[i], v6, pmat, cs01, gembd,
                      btembd, w3, b3)
  return out[:, :N_CLASSES]
